# Initial kernel scaffold; baseline (speedup 1.0000x reference)
import jax
import jax.numpy as jnp
from jax.experimental import pallas as pl


def kernel(hidden_states, gate_w, w1, w2, w3):
    raise NotImplementedError("write your pallas kernel here")


# speedup vs baseline: 1.4931x; 1.4931x over previous
import functools

import jax
import jax.numpy as jnp
from jax.experimental import pallas as pl
from jax.experimental.pallas import tpu as pltpu

_JITTER = 0.01


def _router_body(x_ref, gw_ref, logits_ref, wall_ref):
    x = x_ref[...]
    gw = gw_ref[...]
    logits = jnp.dot(x, gw.T, preferred_element_type=jnp.float32)
    E = logits.shape[1]
    col = jax.lax.broadcasted_iota(jnp.int32, logits.shape, 1)
    neg_inf = jnp.float32(-jnp.inf)
    m1 = jnp.max(logits, axis=1, keepdims=True)
    e1 = jnp.min(jnp.where(logits == m1, col, E), axis=1, keepdims=True)
    masked = jnp.where(col == e1, neg_inf, logits)
    m2 = jnp.max(masked, axis=1, keepdims=True)
    e2 = jnp.min(jnp.where(masked == m2, col, E), axis=1, keepdims=True)
    eps = jnp.float32(2.0 * _JITTER)
    f1 = jnp.maximum(jnp.abs(logits), m1)
    p1 = jax.nn.softmax(jnp.where((m1 - logits) / f1 > eps, neg_inf, logits), axis=-1)
    mult1 = jnp.sum(jnp.where(col == e1, p1, 0.0), axis=1, keepdims=True)
    f2 = jnp.maximum(jnp.abs(logits), m2)
    ssc = jnp.where(col == e1, neg_inf, logits)
    p2 = jax.nn.softmax(jnp.where((m2 - logits) / f2 > eps, neg_inf, ssc), axis=-1)
    mult2 = jnp.sum(jnp.where(col == e2, p2, 0.0), axis=1, keepdims=True)
    logits_ref[...] = logits
    wall_ref[...] = (jnp.where(col == e1, mult1, 0.0)
                     + jnp.where(col == e2, mult2, 0.0))


def _moe_dense_body(wall_ref, x_ref, w1_ref, w2_ref, w3_ref, out_ref):
    e = pl.program_id(0)
    f = pl.program_id(1)
    x = x_ref[...]
    a = jnp.dot(x, w1_ref[0].T, preferred_element_type=jnp.float32)
    b = jnp.dot(x, w3_ref[0].T, preferred_element_type=jnp.float32)
    wall = wall_ref[...]
    ecol = jax.lax.broadcasted_iota(jnp.int32, wall.shape, 1)
    wcol = jnp.sum(jnp.where(ecol == e, wall, 0.0), axis=1, keepdims=True)
    h = (a * jax.nn.sigmoid(a)) * b * wcol
    o = jnp.dot(h, w2_ref[0].T, preferred_element_type=jnp.float32)

    @pl.when((e == 0) & (f == 0))
    def _init():
        out_ref[...] = o

    @pl.when((e > 0) | (f > 0))
    def _acc():
        out_ref[...] += o


def kernel(hidden_states, gate_w, w1, w2, w3, interpret=False):
    b, s, H = hidden_states.shape
    T = b * s
    E, F, _ = w1.shape
    x = hidden_states.reshape(T, H)

    logits, wall = pl.pallas_call(
        _router_body,
        out_shape=[
            jax.ShapeDtypeStruct((T, E), jnp.float32),
            jax.ShapeDtypeStruct((T, E), jnp.float32),
        ],
        interpret=interpret,
    )(x, gate_w)

    FB = min(F, 1024)
    NF = F // FB
    out = pl.pallas_call(
        _moe_dense_body,
        grid=(E, NF),
        in_specs=[
            pl.BlockSpec((T, E), lambda e, f: (0, 0)),
            pl.BlockSpec((T, H), lambda e, f: (0, 0)),
            pl.BlockSpec((1, FB, H), lambda e, f: (e, f, 0)),
            pl.BlockSpec((1, H, FB), lambda e, f: (e, 0, f)),
            pl.BlockSpec((1, FB, H), lambda e, f: (e, f, 0)),
        ],
        out_specs=pl.BlockSpec((T, H), lambda e, f: (0, 0)),
        out_shape=jax.ShapeDtypeStruct((T, H), jnp.float32),
        compiler_params=pltpu.CompilerParams(
            dimension_semantics=("arbitrary", "arbitrary"),
        ),
        interpret=interpret,
    )(wall, x, w1, w2, w3)

    return out.reshape(b, s, H), logits


# speedup vs baseline: 1.5017x; 1.0058x over previous
import functools

import jax
import jax.numpy as jnp
from jax import lax
from jax.experimental import pallas as pl
from jax.experimental.pallas import tpu as pltpu
from jax.experimental.pallas import tpu_sc as plsc

_JITTER = 0.01
_TM = 256
_FB = 1024
_NW = 32
_CHUNK = 32


def _router_body(tm, x_ref, gw_ref, logits_ref, pos1_ref, pos2_ref,
                 wa1_ref, wa2_ref, te_ref, na_ref):
    T, H = x_ref.shape
    E = gw_ref.shape[0]
    x = x_ref[...]
    logits = jnp.dot(x, gw_ref[...].T, preferred_element_type=jnp.float32)
    col = lax.broadcasted_iota(jnp.int32, (T, E), 1)
    neg_inf = jnp.float32(-jnp.inf)
    m1 = jnp.max(logits, axis=1, keepdims=True)
    e1 = jnp.min(jnp.where(logits == m1, col, E), axis=1, keepdims=True)
    masked = jnp.where(col == e1, neg_inf, logits)
    m2 = jnp.max(masked, axis=1, keepdims=True)
    e2 = jnp.min(jnp.where(masked == m2, col, E), axis=1, keepdims=True)
    eps = jnp.float32(2.0 * _JITTER)
    f1 = jnp.maximum(jnp.abs(logits), m1)
    p1 = jax.nn.softmax(jnp.where((m1 - logits) / f1 > eps, neg_inf, logits), axis=-1)
    mult1 = jnp.sum(jnp.where(col == e1, p1, 0.0), axis=1, keepdims=True)
    f2 = jnp.maximum(jnp.abs(logits), m2)
    ssc = jnp.where(col == e1, neg_inf, logits)
    p2 = jax.nn.softmax(jnp.where((m2 - logits) / f2 > eps, neg_inf, ssc), axis=-1)
    mult2 = jnp.sum(jnp.where(col == e2, p2, 0.0), axis=1, keepdims=True)

    onehot1 = (col == e1).astype(jnp.float32)
    onehot2 = (col == e2).astype(jnp.float32)
    ri = lax.broadcasted_iota(jnp.int32, (T, T), 0)
    ci = lax.broadcasted_iota(jnp.int32, (T, T), 1)
    tri = (ci < ri).astype(jnp.float32)
    c1 = jnp.dot(tri, onehot1, preferred_element_type=jnp.float32)
    c2 = jnp.dot(tri, onehot2, preferred_element_type=jnp.float32)
    g1 = jnp.sum(onehot1, axis=0, keepdims=True)
    g2 = jnp.sum(onehot2, axis=0, keepdims=True)
    g = g1 + g2
    gp = jnp.floor((g + (tm - 1)) / tm) * tm
    col8 = lax.broadcasted_iota(jnp.int32, (1, E), 1)
    poff = jnp.zeros((1, E), jnp.float32)
    for ep in range(E):
        gp_e = jnp.sum(jnp.where(col8 == ep, gp, 0.0), axis=1, keepdims=True)
        poff = poff + jnp.where(col8 > ep, gp_e, 0.0)
    pend = poff + gp
    pos1 = jnp.sum((poff + c1) * onehot1, axis=1, keepdims=True)
    pos2 = jnp.sum((poff + g1 + c2) * onehot2, axis=1, keepdims=True)

    ti = lax.broadcasted_iota(jnp.int32, (1, 128), 1).astype(jnp.float32)
    te = jnp.zeros((1, 128), jnp.float32)
    for ep in range(E):
        pend_e = jnp.sum(jnp.where(col8 == ep, pend, 0.0), axis=1, keepdims=True)
        te = te + (ti * tm >= pend_e).astype(jnp.float32)
    ptot = jnp.sum(jnp.where(col8 == E - 1, pend, 0.0), axis=1, keepdims=True)

    logits_ref[...] = logits
    pos1_ref[...] = pos1.astype(jnp.int32)
    pos2_ref[...] = pos2.astype(jnp.int32)
    wa1_ref[...] = mult1
    wa2_ref[...] = mult2
    te_ref[...] = jnp.minimum(te, E - 1).astype(jnp.int32)
    na_ref[...] = (ptot / tm).astype(jnp.int32)


def _run_router(x, gate_w, tm):
    T, _ = x.shape
    E = gate_w.shape[0]
    return pl.pallas_call(
        functools.partial(_router_body, tm),
        out_shape=[
            jax.ShapeDtypeStruct((T, E), jnp.float32),
            jax.ShapeDtypeStruct((T, 1), jnp.int32),
            jax.ShapeDtypeStruct((T, 1), jnp.int32),
            jax.ShapeDtypeStruct((T, 1), jnp.float32),
            jax.ShapeDtypeStruct((T, 1), jnp.float32),
            jax.ShapeDtypeStruct((1, 128), jnp.int32),
            jax.ShapeDtypeStruct((1, 1), jnp.int32),
        ],
    )(x, gate_w)


def _sc_scatter_body(T, half, nch, x_hbm, pos1_hbm, pos2_hbm, xs_hbm,
                     xbuf, idx, sem):
    wid = lax.axis_index("s") * 2 + lax.axis_index("c")

    def do_slot(pos_hbm, base):
        for c in range(nch):
            tb = base + c * _CHUNK
            pltpu.sync_copy(x_hbm.at[pl.ds(tb, _CHUNK)], xbuf)
            pltpu.sync_copy(pos_hbm.at[pl.ds(tb, _CHUNK)], idx)
            pltpu.async_copy(xbuf, xs_hbm.at[idx], sem).wait()

    @pl.when(wid < half)
    def _():
        do_slot(pos1_hbm, wid * nch * _CHUNK)

    @pl.when(wid >= half)
    def _():
        do_slot(pos2_hbm, (wid - half) * nch * _CHUNK)


def _run_scatter(x, pos1, pos2, P):
    T, H = x.shape
    half = _NW // 2
    nch = T // (half * _CHUNK)
    mesh = plsc.VectorSubcoreMesh(core_axis_name="c", subcore_axis_name="s")
    fn = pl.kernel(
        functools.partial(_sc_scatter_body, T, half, nch),
        mesh=mesh,
        out_type=jax.ShapeDtypeStruct((P, H), jnp.float32),
        scratch_types=[
            pltpu.VMEM((_CHUNK, H), jnp.float32),
            pltpu.VMEM((_CHUNK,), jnp.int32),
            pltpu.SemaphoreType.DMA,
        ],
    )
    return fn(x, pos1, pos2)


def _group_body(te_ref, na_ref, x_ref, w1_ref, w3_ref, w2_ref, out_ref):
    i = pl.program_id(0)
    f = pl.program_id(1)

    @pl.when(i < na_ref[0])
    def _():
        x = x_ref[...]
        a = jnp.dot(x, w1_ref[0].T, preferred_element_type=jnp.float32)
        b = jnp.dot(x, w3_ref[0].T, preferred_element_type=jnp.float32)
        h = a * jax.nn.sigmoid(a) * b
        o = jnp.dot(h, w2_ref[0].T, preferred_element_type=jnp.float32)

        @pl.when(f == 0)
        def _init():
            out_ref[...] = o

        @pl.when(f > 0)
        def _acc():
            out_ref[...] += o


def _run_group(te_arr, na_arr, x_s, w1, w3, w2, NT):
    P, H = x_s.shape
    E, F, _ = w1.shape
    NF = F // _FB
    grid_spec = pltpu.PrefetchScalarGridSpec(
        num_scalar_prefetch=2,
        grid=(NT, NF),
        in_specs=[
            pl.BlockSpec((_TM, H), lambda i, f, te, na: (i, 0)),
            pl.BlockSpec((1, _FB, H), lambda i, f, te, na: (te[i], f, 0)),
            pl.BlockSpec((1, _FB, H), lambda i, f, te, na: (te[i], f, 0)),
            pl.BlockSpec((1, H, _FB), lambda i, f, te, na: (te[i], 0, f)),
        ],
        out_specs=pl.BlockSpec((_TM, H), lambda i, f, te, na: (i, 0)),
    )
    return pl.pallas_call(
        _group_body,
        grid_spec=grid_spec,
        out_shape=jax.ShapeDtypeStruct((P, H), jnp.float32),
        compiler_params=pltpu.CompilerParams(
            dimension_semantics=("arbitrary", "arbitrary"),
        ),
    )(te_arr, na_arr, x_s, w1, w3, w2)


def _sc_gather_body(T, nch, outs_hbm, pos1_hbm, pos2_hbm, g1_hbm, g2_hbm,
                    buf, idx, sem):
    wid = lax.axis_index("s") * 2 + lax.axis_index("c")
    base = wid * nch * _CHUNK
    for c in range(nch):
        tb = base + c * _CHUNK
        pltpu.sync_copy(pos1_hbm.at[pl.ds(tb, _CHUNK)], idx)
        pltpu.async_copy(outs_hbm.at[idx], buf, sem).wait()
        pltpu.sync_copy(buf, g1_hbm.at[pl.ds(tb, _CHUNK)])
        pltpu.sync_copy(pos2_hbm.at[pl.ds(tb, _CHUNK)], idx)
        pltpu.async_copy(outs_hbm.at[idx], buf, sem).wait()
        pltpu.sync_copy(buf, g2_hbm.at[pl.ds(tb, _CHUNK)])


def _run_gather(out_s, pos1, pos2, T):
    P, H = out_s.shape
    nch = T // (_NW * _CHUNK)
    mesh = plsc.VectorSubcoreMesh(core_axis_name="c", subcore_axis_name="s")
    fn = pl.kernel(
        functools.partial(_sc_gather_body, T, nch),
        mesh=mesh,
        out_type=[
            jax.ShapeDtypeStruct((T, H), jnp.float32),
            jax.ShapeDtypeStruct((T, H), jnp.float32),
        ],
        scratch_types=[
            pltpu.VMEM((_CHUNK, H), jnp.float32),
            pltpu.VMEM((_CHUNK,), jnp.int32),
            pltpu.SemaphoreType.DMA,
        ],
    )
    return fn(out_s, pos1, pos2)


def _combine_body(g1_ref, g2_ref, wa1_ref, wa2_ref, out_ref):
    out_ref[...] = wa1_ref[...] * g1_ref[...] + wa2_ref[...] * g2_ref[...]


def _run_combine(g1, g2, wa1, wa2):
    T, H = g1.shape
    return pl.pallas_call(
        _combine_body,
        out_shape=jax.ShapeDtypeStruct((T, H), jnp.float32),
    )(g1, g2, wa1, wa2)


def kernel(hidden_states, gate_w, w1, w2, w3):
    b, s, H = hidden_states.shape
    T = b * s
    E, F, _ = w1.shape
    x = hidden_states.reshape(T, H)
    NT = (2 * T + E * (_TM - 1)) // _TM
    P = NT * _TM

    logits, pos1, pos2, wa1, wa2, te, na = _run_router(x, gate_w, _TM)
    pos1f = pos1.reshape(T)
    pos2f = pos2.reshape(T)
    x_s = _run_scatter(x, pos1f, pos2f, P)
    out_s = _run_group(te.reshape(-1), na.reshape(-1), x_s, w1, w3, w2, NT)
    g1, g2 = _run_gather(out_s, pos1f, pos2f, T)
    final = _run_combine(g1, g2, wa1, wa2)
    return final.reshape(b, s, H), logits
